# baseline (speedup 1.0000x reference)
import functools

import jax
import jax.numpy as jnp
from jax import lax
from jax.experimental import pallas as pl
from jax.experimental.pallas import tpu as pltpu
from jax.experimental.pallas import tpu_sc as plsc

_N = 10000
_D_IN = 128
_F = 64
_NACC = 10240
_NW = 32
_EPW = 10368
_EP = _EPW * _NW
_NSUP = _EPW // 384
_ROWS = _NACC // 16

_HIGH = lax.Precision.DEFAULT


def _prep1_body(x_ref, w1_ref, as_ref, ad_ref, h_ref, a_s_ref, a_d_ref):
    h = jnp.dot(x_ref[...], w1_ref[...], precision=_HIGH)
    h_ref[...] = h
    a_s_ref[...] = jnp.dot(h, as_ref[...], precision=_HIGH)
    a_d_ref[...] = jnp.dot(h, ad_ref[...], precision=_HIGH)


def _prep1(x, W1, As1, Ad1):
    blk = 2000
    grid = _N // blk
    return pl.pallas_call(
        _prep1_body,
        grid=(grid,),
        in_specs=[
            pl.BlockSpec((blk, _D_IN), lambda i: (i, 0)),
            pl.BlockSpec((_D_IN, _F), lambda i: (0, 0)),
            pl.BlockSpec((_F, 16), lambda i: (0, 0)),
            pl.BlockSpec((_F, 16), lambda i: (0, 0)),
        ],
        out_specs=[
            pl.BlockSpec((blk, _F), lambda i: (i, 0)),
            pl.BlockSpec((blk, 16), lambda i: (i, 0)),
            pl.BlockSpec((blk, 16), lambda i: (i, 0)),
        ],
        out_shape=[
            jax.ShapeDtypeStruct((_N, _F), jnp.float32),
            jax.ShapeDtypeStruct((_N, 16), jnp.float32),
            jax.ShapeDtypeStruct((_N, 16), jnp.float32),
        ],
    )(x, W1, As1, Ad1)


def _mid_body(na_ref, nb_ref, da_ref, db_ref, r_ref, b_ref, w2_ref,
              a2s_ref, a2d_ref, h_ref, a_s_ref, a_d_ref):
    num = na_ref[...] + nb_ref[...]
    den = da_ref[..., :8] + db_ref[..., :8]
    den_rep = jnp.dot(den, r_ref[...], precision=_HIGH)
    o = jnp.maximum(num / (den_rep + 1e-16) + b_ref[...], 0.0)
    h2 = jnp.dot(o, w2_ref[...], precision=_HIGH)
    h_ref[...] = h2
    a_s_ref[...] = jnp.dot(h2, a2s_ref[...], precision=_HIGH)
    a_d_ref[...] = jnp.dot(h2, a2d_ref[...], precision=_HIGH)


def _mid(na, nb, da, db, R, b1, W2, A2s, A2d):
    blk = 2048
    grid = _NACC // blk
    return pl.pallas_call(
        _mid_body,
        grid=(grid,),
        in_specs=[
            pl.BlockSpec((blk, _F), lambda i: (i, 0)),
            pl.BlockSpec((blk, _F), lambda i: (i, 0)),
            pl.BlockSpec((blk, 16), lambda i: (i, 0)),
            pl.BlockSpec((blk, 16), lambda i: (i, 0)),
            pl.BlockSpec((8, _F), lambda i: (0, 0)),
            pl.BlockSpec((1, _F), lambda i: (0, 0)),
            pl.BlockSpec((_F, _F), lambda i: (0, 0)),
            pl.BlockSpec((_F, 16), lambda i: (0, 0)),
            pl.BlockSpec((_F, 16), lambda i: (0, 0)),
        ],
        out_specs=[
            pl.BlockSpec((blk, _F), lambda i: (i, 0)),
            pl.BlockSpec((blk, 16), lambda i: (i, 0)),
            pl.BlockSpec((blk, 16), lambda i: (i, 0)),
        ],
        out_shape=[
            jax.ShapeDtypeStruct((_NACC, _F), jnp.float32),
            jax.ShapeDtypeStruct((_NACC, 16), jnp.float32),
            jax.ShapeDtypeStruct((_NACC, 16), jnp.float32),
        ],
    )(na, nb, da, db, R, b1, W2, A2s, A2d)


def _fin_body(na_ref, nb_ref, da_ref, db_ref, r_ref, b_ref, o_ref):
    num = na_ref[...] + nb_ref[...]
    den = da_ref[..., :8] + db_ref[..., :8]
    den_rep = jnp.dot(den, r_ref[...], precision=_HIGH)
    o_ref[...] = jnp.maximum(num / (den_rep + 1e-16) + b_ref[...], 0.0)


def _fin(na, nb, da, db, R, b2):
    blk = 2048
    grid = _NACC // blk
    return pl.pallas_call(
        _fin_body,
        grid=(grid,),
        in_specs=[
            pl.BlockSpec((blk, _F), lambda i: (i, 0)),
            pl.BlockSpec((blk, _F), lambda i: (i, 0)),
            pl.BlockSpec((blk, 16), lambda i: (i, 0)),
            pl.BlockSpec((blk, 16), lambda i: (i, 0)),
            pl.BlockSpec((8, _F), lambda i: (0, 0)),
            pl.BlockSpec((1, _F), lambda i: (0, 0)),
        ],
        out_specs=pl.BlockSpec((blk, _F), lambda i: (i, 0)),
        out_shape=jax.ShapeDtypeStruct((_NACC, _F), jnp.float32),
    )(na, nb, da, db, R, b2)


def _sc_body(h_hbm, as_hbm, ad_hbm, src_hbm, dst_hbm, z64_hbm, z16_hbm,
             num_out, den_out,
             src_v, dst_v, sg, dg, h_g, ex_v, contrib,
             num_sh, den_sh, sem1, sem2, sem3, semn, semd, semn2, semd2):
    c = lax.axis_index("c")
    s = lax.axis_index("s")
    wid = s * 2 + c

    r0 = s * _ROWS
    pltpu.sync_copy(z64_hbm, num_sh.at[pl.ds(r0, _ROWS)])
    pltpu.sync_copy(z16_hbm, den_sh.at[pl.ds(r0, _ROWS)])
    plsc.subcore_barrier()

    base_row = wid * (_EPW // 128)
    iot = lax.iota(jnp.int32, 16)

    pltpu.sync_copy(src_hbm.at[pl.ds(base_row, 3)], src_v.at[0])
    pltpu.sync_copy(dst_hbm.at[pl.ds(base_row, 3)], dst_v.at[0])

    def superstep(t, carry):
        p = lax.rem(t, 2)
        for j in range(3):
            pltpu.async_copy(as_hbm.at[src_v.at[p, j]], sg.at[j], sem1)
            pltpu.async_copy(ad_hbm.at[dst_v.at[p, j]], dg.at[j], sem2)
            pltpu.async_copy(h_hbm.at[src_v.at[p, j]], h_g.at[j], sem3)

        @pl.when(t + 1 < _NSUP)
        def _():
            row = base_row + (t + 1) * 3
            pltpu.sync_copy(src_hbm.at[pl.ds(row, 3)], src_v.at[1 - p])
            pltpu.sync_copy(dst_hbm.at[pl.ds(row, 3)], dst_v.at[1 - p])

        for j in range(3):
            pltpu.make_async_copy(as_hbm.at[src_v.at[p, j]], sg.at[j], sem1).wait()
            pltpu.make_async_copy(ad_hbm.at[dst_v.at[p, j]], dg.at[j], sem2).wait()
            pltpu.make_async_copy(h_hbm.at[src_v.at[p, j]], h_g.at[j], sem3).wait()

            sn = semn if j % 2 == 0 else semn2
            sd = semd if j % 2 == 0 else semd2

            @pl.when(t > 0)
            def _():
                pltpu.make_async_copy(
                    contrib.at[j], num_sh.at[pl.ds(0, 128)], sn).wait()
                pltpu.make_async_copy(
                    ex_v.at[j], den_sh.at[pl.ds(0, 128)], sd).wait()

            def ebody(e, _):
                v = sg[j, e, :] + dg[j, e, :]
                v = jnp.where(v > 0.0, v, 0.2 * v)
                ex = jnp.exp(v)
                ex_v[j, e, :] = ex
                for q in range(4):
                    exv = jnp.where(iot < 8, ex[2 * q], ex[2 * q + 1])
                    contrib[j, e, pl.ds(q * 16, 16)] = (
                        h_g[j, e, pl.ds(q * 16, 16)] * exv)
                return 0

            lax.fori_loop(0, 128, ebody, 0, unroll=2)

            pltpu.async_copy(contrib.at[j], num_sh.at[dst_v.at[p, j]], sn,
                             add=True)
            pltpu.async_copy(ex_v.at[j], den_sh.at[dst_v.at[p, j]], sd,
                             add=True)
        return carry

    lax.fori_loop(0, _NSUP, superstep, 0)

    for j in range(3):
        sn = semn if j % 2 == 0 else semn2
        sd = semd if j % 2 == 0 else semd2
        pltpu.make_async_copy(contrib.at[j], num_sh.at[pl.ds(0, 128)], sn).wait()
        pltpu.make_async_copy(ex_v.at[j], den_sh.at[pl.ds(0, 128)], sd).wait()

    plsc.subcore_barrier()
    pltpu.sync_copy(num_sh.at[pl.ds(r0, _ROWS)], num_out.at[c, pl.ds(r0, _ROWS)])
    pltpu.sync_copy(den_sh.at[pl.ds(r0, _ROWS)], den_out.at[c, pl.ds(r0, _ROWS)])


_sc_edge_pass = pl.kernel(
    _sc_body,
    out_type=[
        jax.ShapeDtypeStruct((2, _NACC, _F), jnp.float32),
        jax.ShapeDtypeStruct((2, _NACC, 16), jnp.float32),
    ],
    mesh=plsc.VectorSubcoreMesh(core_axis_name="c", subcore_axis_name="s"),
    compiler_params=pltpu.CompilerParams(use_tc_tiling_on_sc=False),
    scratch_types=[
        pltpu.VMEM((2, 3, 128), jnp.int32),
        pltpu.VMEM((2, 3, 128), jnp.int32),
        pltpu.VMEM((3, 128, 16), jnp.float32),
        pltpu.VMEM((3, 128, 16), jnp.float32),
        pltpu.VMEM((3, 128, _F), jnp.float32),
        pltpu.VMEM((3, 128, 16), jnp.float32),
        pltpu.VMEM((3, 128, _F), jnp.float32),
        pltpu.VMEM_SHARED((_NACC, _F), jnp.float32),
        pltpu.VMEM_SHARED((_NACC, 16), jnp.float32),
        pltpu.SemaphoreType.DMA,
        pltpu.SemaphoreType.DMA,
        pltpu.SemaphoreType.DMA,
        pltpu.SemaphoreType.DMA,
        pltpu.SemaphoreType.DMA,
        pltpu.SemaphoreType.DMA,
        pltpu.SemaphoreType.DMA,
    ],
)


def kernel(x, W1, a_src1, a_dst1, b1, W2, a_src2, a_dst2, b2, edge_index):
    f32 = jnp.float32
    a1s = a_src1.reshape(8, 8).astype(f32)
    a1d = a_dst1.reshape(8, 8).astype(f32)
    eye8 = jnp.eye(8, dtype=f32)
    As1 = jnp.tile((eye8[:, None, :] * a1s[:, :, None]).reshape(64, 8), (1, 2))
    Ad1 = jnp.tile((eye8[:, None, :] * a1d[:, :, None]).reshape(64, 8), (1, 2))
    A2s = jnp.tile(a_src2.reshape(64, 1).astype(f32), (1, 16))
    A2d = jnp.tile(a_dst2.reshape(64, 1).astype(f32), (1, 16))
    R = jnp.repeat(eye8, 8, axis=1)

    loop = jnp.arange(_N, dtype=jnp.int32)
    e_real = edge_index.shape[1] + _N
    pad = _EP - e_real
    src = jnp.concatenate(
        [edge_index[0].astype(jnp.int32), loop, jnp.zeros((pad,), jnp.int32)])
    dst = jnp.concatenate(
        [edge_index[1].astype(jnp.int32), loop, jnp.full((pad,), _N, jnp.int32)])
    src2d = src.reshape(_EP // 128, 128)
    dst2d = dst.reshape(_EP // 128, 128)
    z64 = jnp.zeros((_ROWS, _F), f32)
    z16 = jnp.zeros((_ROWS, 16), f32)

    h1, as1, ad1 = _prep1(x, W1, As1, Ad1)
    num1, den1 = _sc_edge_pass(h1, as1, ad1, src2d, dst2d, z64, z16)
    h2, as2, ad2 = _mid(num1[0], num1[1], den1[0], den1[1], R,
                        b1.reshape(1, _F), W2, A2s, A2d)
    num2, den2 = _sc_edge_pass(h2, as2, ad2, src2d, dst2d, z64, z16)
    out = _fin(num2[0], num2[1], den2[0], den2[1], R, b2.reshape(1, _F))
    return out[:_N]


# speedup vs baseline: 1.0132x; 1.0132x over previous
import functools

import jax
import jax.numpy as jnp
from jax import lax
from jax.experimental import pallas as pl
from jax.experimental.pallas import tpu as pltpu
from jax.experimental.pallas import tpu_sc as plsc

_N = 10000
_D_IN = 128
_F = 64
_NACC = 10240
_NW = 32
_EPW = 10368
_EP = _EPW * _NW
_NSUP = _EPW // 384
_ROWS = _NACC // 16

_HIGH = lax.Precision.DEFAULT


def _prep1_body(x_ref, w1_ref, as_ref, ad_ref, h_ref, a_s_ref, a_d_ref):
    h = jnp.dot(x_ref[...], w1_ref[...], precision=_HIGH)
    h_ref[...] = h
    a_s_ref[...] = jnp.dot(h, as_ref[...], precision=_HIGH)
    a_d_ref[...] = jnp.dot(h, ad_ref[...], precision=_HIGH)


def _prep1(x, W1, As1, Ad1):
    blk = 2000
    grid = _N // blk
    return pl.pallas_call(
        _prep1_body,
        grid=(grid,),
        in_specs=[
            pl.BlockSpec((blk, _D_IN), lambda i: (i, 0)),
            pl.BlockSpec((_D_IN, _F), lambda i: (0, 0)),
            pl.BlockSpec((_F, 16), lambda i: (0, 0)),
            pl.BlockSpec((_F, 16), lambda i: (0, 0)),
        ],
        out_specs=[
            pl.BlockSpec((blk, _F), lambda i: (i, 0)),
            pl.BlockSpec((blk, 16), lambda i: (i, 0)),
            pl.BlockSpec((blk, 16), lambda i: (i, 0)),
        ],
        out_shape=[
            jax.ShapeDtypeStruct((_N, _F), jnp.float32),
            jax.ShapeDtypeStruct((_N, 16), jnp.float32),
            jax.ShapeDtypeStruct((_N, 16), jnp.float32),
        ],
    )(x, W1, As1, Ad1)


def _mid_body(na_ref, nb_ref, da_ref, db_ref, r_ref, b_ref, w2_ref,
              a2s_ref, a2d_ref, h_ref, a_s_ref, a_d_ref):
    num = na_ref[...] + nb_ref[...]
    den = da_ref[..., :8] + db_ref[..., :8]
    den_rep = jnp.dot(den, r_ref[...], precision=_HIGH)
    o = jnp.maximum(num / (den_rep + 1e-16) + b_ref[...], 0.0)
    h2 = jnp.dot(o, w2_ref[...], precision=_HIGH)
    h_ref[...] = h2
    a_s_ref[...] = jnp.dot(h2, a2s_ref[...], precision=_HIGH)
    a_d_ref[...] = jnp.dot(h2, a2d_ref[...], precision=_HIGH)


def _mid(na, nb, da, db, R, b1, W2, A2s, A2d):
    blk = 2048
    grid = _NACC // blk
    return pl.pallas_call(
        _mid_body,
        grid=(grid,),
        in_specs=[
            pl.BlockSpec((blk, _F), lambda i: (i, 0)),
            pl.BlockSpec((blk, _F), lambda i: (i, 0)),
            pl.BlockSpec((blk, 16), lambda i: (i, 0)),
            pl.BlockSpec((blk, 16), lambda i: (i, 0)),
            pl.BlockSpec((8, _F), lambda i: (0, 0)),
            pl.BlockSpec((1, _F), lambda i: (0, 0)),
            pl.BlockSpec((_F, _F), lambda i: (0, 0)),
            pl.BlockSpec((_F, 16), lambda i: (0, 0)),
            pl.BlockSpec((_F, 16), lambda i: (0, 0)),
        ],
        out_specs=[
            pl.BlockSpec((blk, _F), lambda i: (i, 0)),
            pl.BlockSpec((blk, 16), lambda i: (i, 0)),
            pl.BlockSpec((blk, 16), lambda i: (i, 0)),
        ],
        out_shape=[
            jax.ShapeDtypeStruct((_NACC, _F), jnp.float32),
            jax.ShapeDtypeStruct((_NACC, 16), jnp.float32),
            jax.ShapeDtypeStruct((_NACC, 16), jnp.float32),
        ],
    )(na, nb, da, db, R, b1, W2, A2s, A2d)


def _fin_body(na_ref, nb_ref, da_ref, db_ref, r_ref, b_ref, o_ref):
    num = na_ref[...] + nb_ref[...]
    den = da_ref[..., :8] + db_ref[..., :8]
    den_rep = jnp.dot(den, r_ref[...], precision=_HIGH)
    o_ref[...] = jnp.maximum(num / (den_rep + 1e-16) + b_ref[...], 0.0)


def _fin(na, nb, da, db, R, b2):
    blk = 2048
    grid = _NACC // blk
    return pl.pallas_call(
        _fin_body,
        grid=(grid,),
        in_specs=[
            pl.BlockSpec((blk, _F), lambda i: (i, 0)),
            pl.BlockSpec((blk, _F), lambda i: (i, 0)),
            pl.BlockSpec((blk, 16), lambda i: (i, 0)),
            pl.BlockSpec((blk, 16), lambda i: (i, 0)),
            pl.BlockSpec((8, _F), lambda i: (0, 0)),
            pl.BlockSpec((1, _F), lambda i: (0, 0)),
        ],
        out_specs=pl.BlockSpec((blk, _F), lambda i: (i, 0)),
        out_shape=jax.ShapeDtypeStruct((_NACC, _F), jnp.float32),
    )(na, nb, da, db, R, b2)


def _sc_body(h_hbm, as_hbm, ad_hbm, src_hbm, dst_hbm, z64_hbm, z16_hbm,
             num_out, den_out,
             src_v, dst_v, sg, dg, h_g, ex_v, contrib,
             num_sh, den_sh, sem1, sem2, sem3, semn, semd):
    c = lax.axis_index("c")
    s = lax.axis_index("s")
    wid = s * 2 + c

    r0 = s * _ROWS
    pltpu.sync_copy(z64_hbm, num_sh.at[pl.ds(r0, _ROWS)])
    pltpu.sync_copy(z16_hbm, den_sh.at[pl.ds(r0, _ROWS)])
    plsc.subcore_barrier()

    base_row = wid * (_EPW // 128)
    iot = lax.iota(jnp.int32, 16)

    pltpu.sync_copy(src_hbm.at[pl.ds(base_row, 3)], src_v.at[0])
    pltpu.sync_copy(dst_hbm.at[pl.ds(base_row, 3)], dst_v.at[0])

    def superstep(t, carry):
        p = lax.rem(t, 2)
        for j in range(3):
            pltpu.async_copy(as_hbm.at[src_v.at[p, j]], sg.at[j], sem1)
            pltpu.async_copy(ad_hbm.at[dst_v.at[p, j]], dg.at[j], sem2)
            pltpu.async_copy(h_hbm.at[src_v.at[p, j]], h_g.at[j], sem3)

        @pl.when(t + 1 < _NSUP)
        def _():
            row = base_row + (t + 1) * 3
            pltpu.sync_copy(src_hbm.at[pl.ds(row, 3)], src_v.at[1 - p])
            pltpu.sync_copy(dst_hbm.at[pl.ds(row, 3)], dst_v.at[1 - p])

        for j in range(3):
            pltpu.make_async_copy(as_hbm.at[src_v.at[p, j]], sg.at[j], sem1).wait()
            pltpu.make_async_copy(ad_hbm.at[dst_v.at[p, j]], dg.at[j], sem2).wait()
            pltpu.make_async_copy(h_hbm.at[src_v.at[p, j]], h_g.at[j], sem3).wait()

            @pl.when(t > 0)
            def _():
                pltpu.make_async_copy(
                    contrib.at[j], num_sh.at[pl.ds(0, 128)], semn).wait()
                pltpu.make_async_copy(
                    ex_v.at[j], den_sh.at[pl.ds(0, 128)], semd).wait()

            def ebody(e, _):
                v = sg[j, e, :] + dg[j, e, :]
                v = jnp.where(v > 0.0, v, 0.2 * v)
                ex = jnp.exp(v)
                ex_v[j, e, :] = ex
                for q in range(4):
                    exv = jnp.where(iot < 8, ex[2 * q], ex[2 * q + 1])
                    contrib[j, e, pl.ds(q * 16, 16)] = (
                        h_g[j, e, pl.ds(q * 16, 16)] * exv)
                return 0

            lax.fori_loop(0, 128, ebody, 0, unroll=4)

            pltpu.async_copy(contrib.at[j], num_sh.at[dst_v.at[p, j]], semn,
                             add=True)
            pltpu.async_copy(ex_v.at[j], den_sh.at[dst_v.at[p, j]], semd,
                             add=True)
        return carry

    lax.fori_loop(0, _NSUP, superstep, 0)

    for j in range(3):
        pltpu.make_async_copy(contrib.at[j], num_sh.at[pl.ds(0, 128)], semn).wait()
        pltpu.make_async_copy(ex_v.at[j], den_sh.at[pl.ds(0, 128)], semd).wait()

    plsc.subcore_barrier()
    pltpu.sync_copy(num_sh.at[pl.ds(r0, _ROWS)], num_out.at[c, pl.ds(r0, _ROWS)])
    pltpu.sync_copy(den_sh.at[pl.ds(r0, _ROWS)], den_out.at[c, pl.ds(r0, _ROWS)])


_sc_edge_pass = pl.kernel(
    _sc_body,
    out_type=[
        jax.ShapeDtypeStruct((2, _NACC, _F), jnp.float32),
        jax.ShapeDtypeStruct((2, _NACC, 16), jnp.float32),
    ],
    mesh=plsc.VectorSubcoreMesh(core_axis_name="c", subcore_axis_name="s"),
    compiler_params=pltpu.CompilerParams(use_tc_tiling_on_sc=False),
    scratch_types=[
        pltpu.VMEM((2, 3, 128), jnp.int32),
        pltpu.VMEM((2, 3, 128), jnp.int32),
        pltpu.VMEM((3, 128, 16), jnp.float32),
        pltpu.VMEM((3, 128, 16), jnp.float32),
        pltpu.VMEM((3, 128, _F), jnp.float32),
        pltpu.VMEM((3, 128, 16), jnp.float32),
        pltpu.VMEM((3, 128, _F), jnp.float32),
        pltpu.VMEM_SHARED((_NACC, _F), jnp.float32),
        pltpu.VMEM_SHARED((_NACC, 16), jnp.float32),
        pltpu.SemaphoreType.DMA,
        pltpu.SemaphoreType.DMA,
        pltpu.SemaphoreType.DMA,
        pltpu.SemaphoreType.DMA,
        pltpu.SemaphoreType.DMA,
    ],
)


def kernel(x, W1, a_src1, a_dst1, b1, W2, a_src2, a_dst2, b2, edge_index):
    f32 = jnp.float32
    a1s = a_src1.reshape(8, 8).astype(f32)
    a1d = a_dst1.reshape(8, 8).astype(f32)
    eye8 = jnp.eye(8, dtype=f32)
    As1 = jnp.tile((eye8[:, None, :] * a1s[:, :, None]).reshape(64, 8), (1, 2))
    Ad1 = jnp.tile((eye8[:, None, :] * a1d[:, :, None]).reshape(64, 8), (1, 2))
    A2s = jnp.tile(a_src2.reshape(64, 1).astype(f32), (1, 16))
    A2d = jnp.tile(a_dst2.reshape(64, 1).astype(f32), (1, 16))
    R = jnp.repeat(eye8, 8, axis=1)

    loop = jnp.arange(_N, dtype=jnp.int32)
    e_real = edge_index.shape[1] + _N
    pad = _EP - e_real
    src = jnp.concatenate(
        [edge_index[0].astype(jnp.int32), loop, jnp.zeros((pad,), jnp.int32)])
    dst = jnp.concatenate(
        [edge_index[1].astype(jnp.int32), loop, jnp.full((pad,), _N, jnp.int32)])
    src2d = src.reshape(_EP // 128, 128)
    dst2d = dst.reshape(_EP // 128, 128)
    z64 = jnp.zeros((_ROWS, _F), f32)
    z16 = jnp.zeros((_ROWS, 16), f32)

    h1, as1, ad1 = _prep1(x, W1, As1, Ad1)
    num1, den1 = _sc_edge_pass(h1, as1, ad1, src2d, dst2d, z64, z16)
    h2, as2, ad2 = _mid(num1[0], num1[1], den1[0], den1[1], R,
                        b1.reshape(1, _F), W2, A2s, A2d)
    num2, den2 = _sc_edge_pass(h2, as2, ad2, src2d, dst2d, z64, z16)
    out = _fin(num2[0], num2[1], den2[0], den2[1], R, b2.reshape(1, _F))
    return out[:_N]
